# baseline (speedup 1.0000x reference)
import functools

import jax
import jax.numpy as jnp
from jax import lax
from jax.experimental import pallas as pl
from jax.experimental.pallas import tpu as pltpu
from jax.experimental.pallas import tpu_sc as plsc

_EMBED = 32
_BATCH = 16384
_NC = 2
_NS = 16
_NW = _NC * _NS
_BPW = _BATCH // _NW
_CHUNK = 128
_NCH = _BPW // _CHUNK


def _gather_body(user_hbm, item_hbm, utab_hbm, itab_hbm, uout_hbm, iout_hbm,
                 uidx_v, iidx_v, urows_v, irows_v, usem, isem):
    wid = lax.axis_index("s") * _NC + lax.axis_index("c")
    base = wid * _BPW
    pltpu.sync_copy(user_hbm.at[pl.ds(base, _BPW)], uidx_v)
    pltpu.sync_copy(item_hbm.at[pl.ds(base, _BPW)], iidx_v)
    waits = []
    for j in range(_NCH):
        sl = pl.ds(j * _CHUNK, _CHUNK)
        waits.append(pltpu.async_copy(utab_hbm.at[uidx_v.at[sl]],
                                      urows_v.at[sl], usem))
        waits.append(pltpu.async_copy(itab_hbm.at[iidx_v.at[sl]],
                                      irows_v.at[sl], isem))
    for w in waits:
        w.wait()
    pltpu.sync_copy(urows_v, uout_hbm.at[pl.ds(base, _BPW)])
    pltpu.sync_copy(irows_v, iout_hbm.at[pl.ds(base, _BPW)])


_sc_gather = functools.partial(
    pl.kernel,
    mesh=plsc.VectorSubcoreMesh(core_axis_name="c", subcore_axis_name="s"),
    out_type=(
        jax.ShapeDtypeStruct((_BATCH, _EMBED), jnp.float32),
        jax.ShapeDtypeStruct((_BATCH, _EMBED), jnp.float32),
    ),
    scratch_types=[
        pltpu.VMEM((_BPW,), jnp.int32),
        pltpu.VMEM((_BPW,), jnp.int32),
        pltpu.VMEM((_BPW, _EMBED), jnp.float32),
        pltpu.VMEM((_BPW, _EMBED), jnp.float32),
        pltpu.SemaphoreType.DMA,
        pltpu.SemaphoreType.DMA,
    ],
    compiler_params=pltpu.CompilerParams(use_tc_tiling_on_sc=False),
)(_gather_body)


def _mlp_body(u_ref, i_ref, w1u_ref, w1i_ref, b1_ref, w2_ref, b2_ref,
              w3_ref, b3_ref, w4_ref, b4_ref, o_ref):
    x = jnp.dot(u_ref[...], w1u_ref[...], preferred_element_type=jnp.float32)
    x = x + jnp.dot(i_ref[...], w1i_ref[...], preferred_element_type=jnp.float32)
    x = jnp.maximum(x + b1_ref[...], 0.0)
    x = jnp.dot(x, w2_ref[...], preferred_element_type=jnp.float32)
    x = jnp.maximum(x + b2_ref[...], 0.0)
    x = jnp.dot(x, w3_ref[...], preferred_element_type=jnp.float32)
    x = jnp.maximum(x + b3_ref[...], 0.0)
    x = jnp.sum(x * w4_ref[...], axis=1, keepdims=True) + b4_ref[...]
    o_ref[...] = jax.nn.sigmoid(x)


_BB = 2048


def _mlp(u_emb, i_emb, w1u, w1i, b1, w2, b2, w3, b3, w4t, b4):
    full = lambda shape: pl.BlockSpec(shape, lambda i: (0, 0))
    return pl.pallas_call(
        _mlp_body,
        grid=(_BATCH // _BB,),
        in_specs=[
            pl.BlockSpec((_BB, _EMBED), lambda i: (i, 0)),
            pl.BlockSpec((_BB, _EMBED), lambda i: (i, 0)),
            full((_EMBED, 64)), full((_EMBED, 64)), full((1, 64)),
            full((64, 32)), full((1, 32)),
            full((32, 16)), full((1, 16)),
            full((1, 16)), full((1, 1)),
        ],
        out_specs=pl.BlockSpec((_BB, 1), lambda i: (i, 0)),
        out_shape=jax.ShapeDtypeStruct((_BATCH, 1), jnp.float32),
    )(u_emb, i_emb, w1u, w1i, b1, w2, b2, w3, b3, w4t, b4)


def kernel(user, item, user_table, item_table, W1, b1, W2, b2, W3, b3, W4, b4):
    user = user.astype(jnp.int32)
    item = item.astype(jnp.int32)
    u_emb, i_emb = _sc_gather(user, item, user_table, item_table)
    return _mlp(
        u_emb, i_emb,
        W1[:_EMBED], W1[_EMBED:], b1.reshape(1, 64),
        W2, b2.reshape(1, 32),
        W3, b3.reshape(1, 16),
        W4.reshape(1, 16), b4.reshape(1, 1),
    )


# speedup vs baseline: 3.0849x; 3.0849x over previous
import functools

import jax
import jax.numpy as jnp
from jax import lax
from jax.experimental import pallas as pl
from jax.experimental.pallas import tpu as pltpu
from jax.experimental.pallas import tpu_sc as plsc

_EMBED = 32
_BATCH = 16384
_NC = 2
_NS = 16
_NW = _NC * _NS
_BPW = _BATCH // _NW
_K = 8
_NG = _BPW // _K
_LROWS = (_BATCH // 128) * _EMBED


def _gather_body(user_hbm, item_hbm, utab_hbm, itab_hbm, uout_hbm, iout_hbm,
                 uidx_v, iidx_v, ubuf, ibuf, ustage, istage, usem, isem):
    wid = lax.axis_index("s") * _NC + lax.axis_index("c")
    base = wid * _BPW
    pltpu.sync_copy(user_hbm.at[pl.ds(base, _BPW)], uidx_v)
    pltpu.sync_copy(item_hbm.at[pl.ds(base, _BPW)], iidx_v)
    half0 = lax.iota(jnp.int32, 16)
    half1 = half0 + 16

    def group(g, carry):
        e0 = g * 16
        uvec = uidx_v[pl.ds(e0, 16)]
        ivec = iidx_v[pl.ds(e0, 16)]
        krow = (e0 // 128) * _EMBED
        kcol = jnp.full((16,), e0 & 127, jnp.int32)
        for sub in range(16 // _K):
            for slot in range(_K):
                su = uvec[sub * _K + slot]
                si = ivec[sub * _K + slot]
                uoff = pl.multiple_of((su >> 7) * 128, 128)
                ioff = pl.multiple_of((si >> 7) * 128, 128)
                pltpu.async_copy(utab_hbm.at[:, pl.ds(uoff, 128)],
                                 ubuf.at[pl.ds(slot * _EMBED, _EMBED), :], usem)
                pltpu.async_copy(itab_hbm.at[:, pl.ds(ioff, 128)],
                                 ibuf.at[pl.ds(slot * _EMBED, _EMBED), :], isem)
            for slot in range(_K):
                pltpu.make_async_copy(utab_hbm.at[:, pl.ds(0, 128)],
                                      ubuf.at[pl.ds(slot * _EMBED, _EMBED), :],
                                      usem).wait()
                pltpu.make_async_copy(itab_hbm.at[:, pl.ds(0, 128)],
                                      ibuf.at[pl.ds(slot * _EMBED, _EMBED), :],
                                      isem).wait()
            for slot in range(_K):
                su = uvec[sub * _K + slot]
                si = ivec[sub * _K + slot]
                cu = jnp.full((16,), su & 127, jnp.int32)
                ci = jnp.full((16,), si & 127, jnp.int32)
                col = kcol + (sub * _K + slot)
                for half in (half0, half1):
                    src = half + slot * _EMBED
                    dst = half + krow
                    vu = plsc.load_gather(ubuf, [src, cu])
                    vi = plsc.load_gather(ibuf, [src, ci])
                    plsc.store_scatter(ustage, [dst, col], vu)
                    plsc.store_scatter(istage, [dst, col], vi)
        return carry

    lax.fori_loop(0, _BPW // 16, group, 0)
    lrow = wid * (_BPW // 128) * _EMBED
    pltpu.sync_copy(ustage, uout_hbm.at[pl.ds(lrow, (_BPW // 128) * _EMBED), :])
    pltpu.sync_copy(istage, iout_hbm.at[pl.ds(lrow, (_BPW // 128) * _EMBED), :])


_sc_gather = functools.partial(
    pl.kernel,
    mesh=plsc.VectorSubcoreMesh(core_axis_name="c", subcore_axis_name="s"),
    out_type=(
        jax.ShapeDtypeStruct((_LROWS, 128), jnp.float32),
        jax.ShapeDtypeStruct((_LROWS, 128), jnp.float32),
    ),
    scratch_types=[
        pltpu.VMEM((_BPW,), jnp.int32),
        pltpu.VMEM((_BPW,), jnp.int32),
        pltpu.VMEM((_EMBED * _K, 128), jnp.float32),
        pltpu.VMEM((_EMBED * _K, 128), jnp.float32),
        pltpu.VMEM(((_BPW // 128) * _EMBED, 128), jnp.float32),
        pltpu.VMEM(((_BPW // 128) * _EMBED, 128), jnp.float32),
        pltpu.SemaphoreType.DMA,
        pltpu.SemaphoreType.DMA,
    ],
    compiler_params=pltpu.CompilerParams(needs_layout_passes=False),
)(_gather_body)


def _mlp_body(u_ref, i_ref, w1u_ref, w1i_ref, b1_ref, w2_ref, b2_ref,
              w3_ref, b3_ref, w4_ref, b4_ref, o_ref):
    x = jnp.dot(w1u_ref[...], u_ref[...], preferred_element_type=jnp.float32)
    x = x + jnp.dot(w1i_ref[...], i_ref[...], preferred_element_type=jnp.float32)
    x = jnp.maximum(x + b1_ref[...], 0.0)
    x = jnp.maximum(jnp.dot(w2_ref[...], x, preferred_element_type=jnp.float32)
                    + b2_ref[...], 0.0)
    x = jnp.maximum(jnp.dot(w3_ref[...], x, preferred_element_type=jnp.float32)
                    + b3_ref[...], 0.0)
    x = jnp.dot(w4_ref[...], x, preferred_element_type=jnp.float32) + b4_ref[...]
    o_ref[...] = jax.nn.sigmoid(x)


_GB = 16
_BR = _GB * _EMBED


def _mlp(u_l, i_l, w1u, w1i, b1, w2, b2, w3, b3, w4, b4):
    full = lambda shape: pl.BlockSpec(shape, lambda i: (0, 0))
    return pl.pallas_call(
        _mlp_body,
        grid=(_LROWS // _BR,),
        in_specs=[
            pl.BlockSpec((_BR, 128), lambda i: (i, 0)),
            pl.BlockSpec((_BR, 128), lambda i: (i, 0)),
            full(( _GB * 64, _BR)), full((_GB * 64, _BR)), full((_GB * 64, 1)),
            full((_GB * 32, _GB * 64)), full((_GB * 32, 1)),
            full((_GB * 16, _GB * 32)), full((_GB * 16, 1)),
            full((_GB, _GB * 16)), full((1, 1)),
        ],
        out_specs=pl.BlockSpec((_GB, 128), lambda i: (i, 0)),
        out_shape=jax.ShapeDtypeStruct((_BATCH // 128, 128), jnp.float32),
    )(u_l, i_l, w1u, w1i, b1, w2, b2, w3, b3, w4, b4)


def kernel(user, item, user_table, item_table, W1, b1, W2, b2, W3, b3, W4, b4):
    user = user.astype(jnp.int32)
    item = item.astype(jnp.int32)
    u_l, i_l = _sc_gather(user, item, user_table.T, item_table.T)
    eye = jnp.eye(_GB, dtype=jnp.float32)
    bd = lambda w: jnp.kron(eye, w)
    tile_b = lambda b: jnp.tile(b, (_GB,)).reshape(-1, 1)
    y = _mlp(
        u_l, i_l,
        bd(W1[:_EMBED].T), bd(W1[_EMBED:].T), tile_b(b1),
        bd(W2.T), tile_b(b2),
        bd(W3.T), tile_b(b3),
        bd(W4.T), b4.reshape(1, 1),
    )
    return y.reshape(_BATCH, 1)


# speedup vs baseline: 3.5691x; 1.1570x over previous
import functools

import jax
import jax.numpy as jnp
from jax import lax
from jax.experimental import pallas as pl
from jax.experimental.pallas import tpu as pltpu
from jax.experimental.pallas import tpu_sc as plsc

_EMBED = 32
_BATCH = 16384
_NC = 2
_NS = 16
_NW = _NC * _NS
_BPW = _BATCH // _NW
_K = 8
_NG = _BPW // _K
_LROWS = (_BATCH // 128) * _EMBED


def _gather_body(user_hbm, item_hbm, utab_hbm, itab_hbm, uout_hbm, iout_hbm,
                 uidx_v, iidx_v, buf, ustage, istage,
                 usem0, usem1, isem0, isem1):
    wid = lax.axis_index("s") * _NC + lax.axis_index("c")
    base = wid * _BPW
    pltpu.sync_copy(user_hbm.at[pl.ds(base, _BPW)], uidx_v.at[pl.ds(0, _BPW)])
    pltpu.sync_copy(item_hbm.at[pl.ds(base, _BPW)], iidx_v.at[pl.ds(0, _BPW)])
    half0 = lax.iota(jnp.int32, 16)
    half1 = half0 + 16
    usems = (usem0, usem1)
    isems = (isem0, isem1)

    def fire(uvec, ivec, k):
        su = uvec[8 + k]
        si = ivec[8 + k]
        uoff = pl.multiple_of((su >> 7) * 128, 128)
        ioff = pl.multiple_of((si >> 7) * 128, 128)
        pltpu.async_copy(utab_hbm.at[:, pl.ds(uoff, 128)],
                         buf.at[pl.ds(2 * k * _EMBED, _EMBED), :],
                         usems[k // 4])
        pltpu.async_copy(itab_hbm.at[:, pl.ds(ioff, 128)],
                         buf.at[pl.ds((2 * k + 1) * _EMBED, _EMBED), :],
                         isems[k // 4])

    def drain(bank):
        for _ in range(4):
            pltpu.make_async_copy(utab_hbm.at[:, pl.ds(0, 128)],
                                  buf.at[pl.ds(0, _EMBED), :],
                                  usems[bank]).wait()
            pltpu.make_async_copy(itab_hbm.at[:, pl.ds(0, 128)],
                                  buf.at[pl.ds(0, _EMBED), :],
                                  isems[bank]).wait()

    def extract(uvec, ivec, e0, k):
        su = uvec[k]
        si = ivec[k]
        cu = jnp.full((16,), su & 127, jnp.int32)
        ci = jnp.full((16,), si & 127, jnp.int32)
        e = e0 + k
        col = jnp.full((16,), e & 127, jnp.int32)
        krow = (e // 128) * _EMBED
        for half in (half0, half1):
            dst = half + krow
            vu = plsc.load_gather(buf, [half + 2 * k * _EMBED, cu])
            vi = plsc.load_gather(buf, [half + (2 * k + 1) * _EMBED, ci])
            plsc.store_scatter(ustage, [dst, col], vu)
            plsc.store_scatter(istage, [dst, col], vi)

    uvec0 = uidx_v[pl.ds(0, 16)]
    ivec0 = iidx_v[pl.ds(0, 16)]
    for k in range(8):
        su = uvec0[k]
        si = ivec0[k]
        uoff = pl.multiple_of((su >> 7) * 128, 128)
        ioff = pl.multiple_of((si >> 7) * 128, 128)
        pltpu.async_copy(utab_hbm.at[:, pl.ds(uoff, 128)],
                         buf.at[pl.ds(2 * k * _EMBED, _EMBED), :],
                         usems[k // 4])
        pltpu.async_copy(itab_hbm.at[:, pl.ds(ioff, 128)],
                         buf.at[pl.ds((2 * k + 1) * _EMBED, _EMBED), :],
                         isems[k // 4])

    def step(j, carry):
        e0 = j * 8
        uvec = uidx_v[pl.ds(e0, 16)]
        ivec = iidx_v[pl.ds(e0, 16)]
        last = j == (_BPW // 8 - 1)
        for bank in range(2):
            drain(bank)
            for k in range(4 * bank, 4 * bank + 4):
                extract(uvec, ivec, e0, k)

            @pl.when(jnp.logical_not(last))
            def _():
                for k in range(4 * bank, 4 * bank + 4):
                    fire(uvec, ivec, k)
        return carry

    lax.fori_loop(0, _BPW // 8, step, 0)
    lrow = wid * (_BPW // 128) * _EMBED
    pltpu.sync_copy(ustage, uout_hbm.at[pl.ds(lrow, (_BPW // 128) * _EMBED), :])
    pltpu.sync_copy(istage, iout_hbm.at[pl.ds(lrow, (_BPW // 128) * _EMBED), :])


_sc_gather = functools.partial(
    pl.kernel,
    mesh=plsc.VectorSubcoreMesh(core_axis_name="c", subcore_axis_name="s"),
    out_type=(
        jax.ShapeDtypeStruct((_LROWS, 128), jnp.float32),
        jax.ShapeDtypeStruct((_LROWS, 128), jnp.float32),
    ),
    scratch_types=[
        pltpu.VMEM((_BPW + 16,), jnp.int32),
        pltpu.VMEM((_BPW + 16,), jnp.int32),
        pltpu.VMEM((16 * _EMBED, 128), jnp.float32),
        pltpu.VMEM(((_BPW // 128) * _EMBED, 128), jnp.float32),
        pltpu.VMEM(((_BPW // 128) * _EMBED, 128), jnp.float32),
        pltpu.SemaphoreType.DMA,
        pltpu.SemaphoreType.DMA,
        pltpu.SemaphoreType.DMA,
        pltpu.SemaphoreType.DMA,
    ],
    compiler_params=pltpu.CompilerParams(needs_layout_passes=False),
)(_gather_body)


def _mlp_body(u_ref, i_ref, w1u_ref, w1i_ref, b1_ref, w2_ref, b2_ref,
              w3_ref, b3_ref, w4_ref, b4_ref, o_ref):
    x = jnp.dot(w1u_ref[...], u_ref[...], preferred_element_type=jnp.float32)
    x = x + jnp.dot(w1i_ref[...], i_ref[...], preferred_element_type=jnp.float32)
    x = jnp.maximum(x + b1_ref[...], 0.0)
    x = jnp.maximum(jnp.dot(w2_ref[...], x, preferred_element_type=jnp.float32)
                    + b2_ref[...], 0.0)
    x = jnp.maximum(jnp.dot(w3_ref[...], x, preferred_element_type=jnp.float32)
                    + b3_ref[...], 0.0)
    x = jnp.dot(w4_ref[...], x, preferred_element_type=jnp.float32) + b4_ref[...]
    o_ref[...] = jax.nn.sigmoid(x)


_GB = 16
_BR = _GB * _EMBED


def _mlp(u_l, i_l, w1u, w1i, b1, w2, b2, w3, b3, w4, b4):
    full = lambda shape: pl.BlockSpec(shape, lambda i: (0, 0))
    return pl.pallas_call(
        _mlp_body,
        grid=(_LROWS // _BR,),
        in_specs=[
            pl.BlockSpec((_BR, 128), lambda i: (i, 0)),
            pl.BlockSpec((_BR, 128), lambda i: (i, 0)),
            full(( _GB * 64, _BR)), full((_GB * 64, _BR)), full((_GB * 64, 1)),
            full((_GB * 32, _GB * 64)), full((_GB * 32, 1)),
            full((_GB * 16, _GB * 32)), full((_GB * 16, 1)),
            full((_GB, _GB * 16)), full((1, 1)),
        ],
        out_specs=pl.BlockSpec((_GB, 128), lambda i: (i, 0)),
        out_shape=jax.ShapeDtypeStruct((_BATCH // 128, 128), jnp.float32),
    )(u_l, i_l, w1u, w1i, b1, w2, b2, w3, b3, w4, b4)


def kernel(user, item, user_table, item_table, W1, b1, W2, b2, W3, b3, W4, b4):
    user = user.astype(jnp.int32)
    item = item.astype(jnp.int32)
    u_l, i_l = _sc_gather(user, item, user_table.T, item_table.T)
    eye = jnp.eye(_GB, dtype=jnp.float32)
    bd = lambda w: jnp.kron(eye, w)
    tile_b = lambda b: jnp.tile(b, (_GB,)).reshape(-1, 1)
    y = _mlp(
        u_l, i_l,
        bd(W1[:_EMBED].T), bd(W1[_EMBED:].T), tile_b(b1),
        bd(W2.T), tile_b(b2),
        bd(W3.T), tile_b(b3),
        bd(W4.T), b4.reshape(1, 1),
    )
    return y.reshape(_BATCH, 1)


# speedup vs baseline: 3.6953x; 1.0353x over previous
import functools

import jax
import jax.numpy as jnp
from jax import lax
from jax.experimental import pallas as pl
from jax.experimental.pallas import tpu as pltpu
from jax.experimental.pallas import tpu_sc as plsc

_EMBED = 32
_BATCH = 16384
_NC = 2
_NS = 16
_NW = _NC * _NS
_BPW = _BATCH // _NW
_K = 8
_NG = _BPW // _K
_LROWS = (_BATCH // 128) * _EMBED


def _gather_body(user_hbm, item_hbm, utab_hbm, itab_hbm, uout_hbm, iout_hbm,
                 uidx_v, iidx_v, buf, stage, sem0, sem1, sem2, sem3):
    wid = lax.axis_index("s") * _NC + lax.axis_index("c")
    base = wid * _BPW
    pltpu.sync_copy(user_hbm.at[pl.ds(base, _BPW)], uidx_v.at[pl.ds(0, _BPW)])
    pltpu.sync_copy(item_hbm.at[pl.ds(base, _BPW)], iidx_v.at[pl.ds(0, _BPW)])
    half0 = lax.iota(jnp.int32, 16)
    half1 = half0 + 16
    sems = (sem0, sem1, sem2, sem3)
    lrow = wid * (_BPW // 128) * _EMBED

    def one_table(idx_v, tab_hbm, out_hbm):
        def fire(vec, k):
            s = vec[k]
            off = pl.multiple_of((s >> 7) * 128, 128)
            pltpu.async_copy(tab_hbm.at[:, pl.ds(off, 128)],
                             buf.at[pl.ds(k * _EMBED, _EMBED), :],
                             sems[k // 4])

        def extract(vec, e0, k):
            s = vec[k]
            c = jnp.full((16,), s & 127, jnp.int32)
            e = e0 + k
            col = jnp.full((16,), e & 127, jnp.int32)
            krow = (e // 128) * _EMBED
            for half in (half0, half1):
                v = plsc.load_gather(buf, [half + k * _EMBED, c])
                plsc.store_scatter(stage, [half + krow, col], v)

        vec0 = idx_v[pl.ds(0, 16)]
        for k in range(16):
            fire(vec0, k)

        def step(j, carry):
            e0 = j * 16
            vec = idx_v[pl.ds(e0, 16)]
            vnx = idx_v[pl.ds(e0 + 16, 16)]
            last = j == (_BPW // 16 - 1)
            for bank in range(4):
                for _ in range(4):
                    pltpu.make_async_copy(tab_hbm.at[:, pl.ds(0, 128)],
                                          buf.at[pl.ds(0, _EMBED), :],
                                          sems[bank]).wait()
                for k in range(4 * bank, 4 * bank + 4):
                    extract(vec, e0, k)

                @pl.when(jnp.logical_not(last))
                def _():
                    for k in range(4 * bank, 4 * bank + 4):
                        fire(vnx, k)
            return carry

        lax.fori_loop(0, _BPW // 16, step, 0)
        pltpu.sync_copy(stage, out_hbm.at[pl.ds(lrow, (_BPW // 128) * _EMBED), :])

    one_table(uidx_v, utab_hbm, uout_hbm)
    one_table(iidx_v, itab_hbm, iout_hbm)


_sc_gather = functools.partial(
    pl.kernel,
    mesh=plsc.VectorSubcoreMesh(core_axis_name="c", subcore_axis_name="s"),
    out_type=(
        jax.ShapeDtypeStruct((_LROWS, 128), jnp.float32),
        jax.ShapeDtypeStruct((_LROWS, 128), jnp.float32),
    ),
    scratch_types=[
        pltpu.VMEM((_BPW + 16,), jnp.int32),
        pltpu.VMEM((_BPW + 16,), jnp.int32),
        pltpu.VMEM((16 * _EMBED, 128), jnp.float32),
        pltpu.VMEM(((_BPW // 128) * _EMBED, 128), jnp.float32),
        pltpu.SemaphoreType.DMA,
        pltpu.SemaphoreType.DMA,
        pltpu.SemaphoreType.DMA,
        pltpu.SemaphoreType.DMA,
    ],
    compiler_params=pltpu.CompilerParams(needs_layout_passes=False),
)(_gather_body)


def _mlp_body(u_ref, i_ref, w1u_ref, w1i_ref, b1_ref, w2_ref, b2_ref,
              w3_ref, b3_ref, w4_ref, b4_ref, o_ref):
    x = jnp.dot(w1u_ref[...], u_ref[...], preferred_element_type=jnp.float32)
    x = x + jnp.dot(w1i_ref[...], i_ref[...], preferred_element_type=jnp.float32)
    x = jnp.maximum(x + b1_ref[...], 0.0)
    x = jnp.maximum(jnp.dot(w2_ref[...], x, preferred_element_type=jnp.float32)
                    + b2_ref[...], 0.0)
    x = jnp.maximum(jnp.dot(w3_ref[...], x, preferred_element_type=jnp.float32)
                    + b3_ref[...], 0.0)
    x = jnp.dot(w4_ref[...], x, preferred_element_type=jnp.float32) + b4_ref[...]
    o_ref[...] = jax.nn.sigmoid(x)


_GB = 16
_BR = _GB * _EMBED


def _mlp(u_l, i_l, w1u, w1i, b1, w2, b2, w3, b3, w4, b4):
    full = lambda shape: pl.BlockSpec(shape, lambda i: (0, 0))
    return pl.pallas_call(
        _mlp_body,
        grid=(_LROWS // _BR,),
        in_specs=[
            pl.BlockSpec((_BR, 128), lambda i: (i, 0)),
            pl.BlockSpec((_BR, 128), lambda i: (i, 0)),
            full(( _GB * 64, _BR)), full((_GB * 64, _BR)), full((_GB * 64, 1)),
            full((_GB * 32, _GB * 64)), full((_GB * 32, 1)),
            full((_GB * 16, _GB * 32)), full((_GB * 16, 1)),
            full((_GB, _GB * 16)), full((1, 1)),
        ],
        out_specs=pl.BlockSpec((_GB, 128), lambda i: (i, 0)),
        out_shape=jax.ShapeDtypeStruct((_BATCH // 128, 128), jnp.float32),
    )(u_l, i_l, w1u, w1i, b1, w2, b2, w3, b3, w4, b4)


def kernel(user, item, user_table, item_table, W1, b1, W2, b2, W3, b3, W4, b4):
    user = user.astype(jnp.int32)
    item = item.astype(jnp.int32)
    u_l, i_l = _sc_gather(user, item, user_table.T, item_table.T)
    eye = jnp.eye(_GB, dtype=jnp.float32)
    bd = lambda w: jnp.kron(eye, w)
    tile_b = lambda b: jnp.tile(b, (_GB,)).reshape(-1, 1)
    y = _mlp(
        u_l, i_l,
        bd(W1[:_EMBED].T), bd(W1[_EMBED:].T), tile_b(b1),
        bd(W2.T), tile_b(b2),
        bd(W3.T), tile_b(b3),
        bd(W4.T), b4.reshape(1, 1),
    )
    return y.reshape(_BATCH, 1)
